# baseline (speedup 1.0000x reference)
import functools

import jax
import jax.numpy as jnp
from jax import lax
from jax.experimental import pallas as pl
from jax.experimental.pallas import tpu as pltpu
from jax.experimental.pallas import tpu_sc as plsc

B = 16384
D = 64
NC = 2
NS = 16
L = 16
NW = NC * NS
BPW = B // NW
NCH = BPW // 128
RING = 6
MAXB = BPW

_mesh = plsc.VectorSubcoreMesh(
    core_axis_name="c", subcore_axis_name="s", num_cores=NC, num_subcores=NS
)

_cp = pltpu.CompilerParams(needs_layout_passes=False)


def _stage(hbm3, vmem, w):
    pltpu.sync_copy(hbm3.at[w], vmem)


def _vec_at(ref, pos):
    row = lax.shift_right_logical(pos, 7)
    off = jnp.bitwise_and(pos, 127)
    aligned = jnp.bitwise_and(off, ~15)
    return ref[row, pl.ds(pl.multiple_of(aligned, 16), 16)]


def _splat_at(ref, pos):
    grp = _vec_at(ref, pos)
    lane = jnp.bitwise_and(pos, 15)
    lvec = jnp.zeros((L,), jnp.int32) + lane
    return jnp.take_along_axis(grp, lvec, axis=0)


def _scalar_at(ref, pos):
    return _splat_at(ref, pos)[0]


def _dvecs():
    base = lax.iota(jnp.int32, L)
    return [base + k * L for k in range(4)]


@functools.partial(
    pl.kernel,
    mesh=_mesh,
    out_type=jax.ShapeDtypeStruct((B, 128), jnp.float32),
    scratch_types=[
        pltpu.VMEM((NCH, 128), jnp.int32),
        pltpu.VMEM((NCH + 1, 128), jnp.int32),
        pltpu.VMEM((NCH, 128), jnp.int32),
        pltpu.VMEM((1, 128), jnp.int32),
        pltpu.VMEM((NCH, 128), jnp.int32),
        pltpu.VMEM((BPW, 128), jnp.float32),
    ]
    + [pltpu.VMEM((D, 128), jnp.float32) for _ in range(RING)]
    + [pltpu.SemaphoreType.DMA for _ in range(RING + 1)],
    compiler_params=_cp,
)
def _phase_a(blist_h, starts_h, cols_h, params_h, opos_h, tabT_h, emb_out,
             blist_v, starts_v, cols_v, params_v, opos_v, emat_v, *ring):
    bufs, sems, ssem = ring[:RING], ring[RING:-1], ring[-1]
    w = lax.axis_index("s") * NC + lax.axis_index("c")
    _stage(blist_h, blist_v, w)
    _stage(starts_h, starts_v, w)
    _stage(cols_h, cols_v, w)
    _stage(params_h, params_v, w)
    _stage(opos_h, opos_v, w)
    nblk = params_v[0, pl.ds(0, 16)][0]
    dvecs = _dvecs()

    def fire(j, r):
        ub = _scalar_at(blist_v, j)
        return pltpu.async_copy(
            tabT_h.at[:, pl.ds(pl.multiple_of(ub * 128, 128), 128)],
            bufs[r], sems[r])

    for r in range(RING):
        @pl.when(r < nblk)
        def _(r=r):
            fire(r, r)

    def extract(j, r):
        s = _scalar_at(starts_v, j)
        t = _scalar_at(starts_v, j + 1)

        def elem(e, _):
            csplat = _splat_at(cols_v, e)
            for k in range(4):
                v = plsc.load_gather(bufs[r], [dvecs[k], csplat])
                emat_v[e, pl.ds(k * L, L)] = v
            return 0

        lax.fori_loop(s, t, elem, 0)

    def outer(j6, _):
        for r in range(RING):
            j = j6 * RING + r

            @pl.when(j < nblk)
            def _(j=j, r=r):
                pltpu.make_async_copy(
                    tabT_h.at[:, pl.ds(0, 128)], bufs[r], sems[r]).wait()
                extract(j, r)

                @pl.when(j + RING < nblk)
                def _(j=j, r=r):
                    fire(j + RING, r)
        return 0

    nrounds = lax.div(nblk + RING - 1, RING)
    lax.fori_loop(0, nrounds, outer, 0)

    scopies = []
    for j in range(NCH):
        scopies.append(pltpu.async_copy(
            emat_v.at[pl.ds(j * 128, 128)], emb_out.at[opos_v.at[j]], ssem))
    for c in scopies:
        c.wait()


@functools.partial(
    pl.kernel,
    mesh=_mesh,
    out_type=jax.ShapeDtypeStruct((B,), jnp.float32),
    scratch_types=[
        pltpu.VMEM((NCH, 128), jnp.int32),
        pltpu.VMEM((NCH + 1, 128), jnp.int32),
        pltpu.VMEM((NCH, 128), jnp.int32),
        pltpu.VMEM((1, 128), jnp.int32),
        pltpu.VMEM((NCH, 128), jnp.int32),
        pltpu.VMEM((BPW, 128), jnp.float32),
        pltpu.VMEM((BPW,), jnp.float32),
    ]
    + [pltpu.VMEM((D, 128), jnp.float32) for _ in range(RING)]
    + [pltpu.SemaphoreType.DMA for _ in range(RING + 1)],
    compiler_params=_cp,
)
def _phase_b(blist_h, starts_h, cols_h, params_h, outpos_h,
             tabT_h, uemb_h, out_h,
             blist_v, starts_v, cols_v, params_v, outpos_v,
             urows_v, outv, *ring):
    bufs, sems, gsem = ring[:RING], ring[RING:-1], ring[-1]
    w = lax.axis_index("s") * NC + lax.axis_index("c")
    _stage(blist_h, blist_v, w)
    _stage(starts_h, starts_v, w)
    _stage(cols_h, cols_v, w)
    _stage(params_h, params_v, w)
    _stage(outpos_h, outpos_v, w)
    nblk = params_v[0, pl.ds(0, 16)][0]
    dvecs = _dvecs()

    def fire(j, r):
        ib = _scalar_at(blist_v, j)
        return pltpu.async_copy(
            tabT_h.at[:, pl.ds(pl.multiple_of(ib * 128, 128), 128)],
            bufs[r], sems[r])

    for r in range(RING):
        @pl.when(r < nblk)
        def _(r=r):
            fire(r, r)

    gcopies = []
    for j in range(NCH):
        gcopies.append(pltpu.async_copy(
            uemb_h.at[outpos_v.at[j]], urows_v.at[pl.ds(j * 128, 128)], gsem))
    for c in gcopies:
        c.wait()

    def extract(j, r):
        s = _scalar_at(starts_v, j)
        t = _scalar_at(starts_v, j + 1)

        def elem(e, _):
            csplat = _splat_at(cols_v, e)
            acc = jnp.zeros((L,), jnp.float32)
            for k in range(4):
                iv = plsc.load_gather(bufs[r], [dvecs[k], csplat])
                uv = urows_v[e, pl.ds(k * L, L)]
                acc = acc + iv * uv
            for sh in (8, 4, 2, 1):
                perm = jnp.bitwise_xor(lax.iota(jnp.int32, L), sh)
                acc = acc + jnp.take_along_axis(acc, perm, axis=0)
            aligned = jnp.bitwise_and(e, ~15)
            lane = jnp.bitwise_and(e, 15)
            lmask = lax.iota(jnp.int32, L) == lane
            cur = outv[pl.ds(pl.multiple_of(aligned, 16), L)]
            outv[pl.ds(pl.multiple_of(aligned, 16), L)] = jnp.where(
                lmask, acc, cur)
            return 0

        lax.fori_loop(s, t, elem, 0)

    def outer(j6, _):
        for r in range(RING):
            j = j6 * RING + r

            @pl.when(j < nblk)
            def _(j=j, r=r):
                pltpu.make_async_copy(
                    tabT_h.at[:, pl.ds(0, 128)], bufs[r], sems[r]).wait()
                extract(j, r)

                @pl.when(j + RING < nblk)
                def _(j=j, r=r):
                    fire(j + RING, r)
        return 0

    nrounds = lax.div(nblk + RING - 1, RING)
    lax.fori_loop(0, nrounds, outer, 0)

    scopies = []
    for j in range(NCH):
        scopies.append(pltpu.async_copy(
            outv.at[pl.ds(j * 128, 128)], out_h.at[outpos_v.at[j]], gsem))
    for c in scopies:
        c.wait()


def _dedup_prep(ids_sorted):
    blocks = lax.shift_right_logical(ids_sorted, 7).reshape(NW, BPW)
    cols = jnp.bitwise_and(ids_sorted, 127).reshape(NW, BPW)
    first = jnp.concatenate(
        [jnp.ones((NW, 1), bool), blocks[:, 1:] != blocks[:, :-1]], axis=1)
    slot = jnp.cumsum(first.astype(jnp.int32), axis=1) - 1
    nblk = slot[:, -1] + 1
    jgrid = jnp.arange(MAXB + 1, dtype=jnp.int32)
    starts = jax.vmap(
        lambda s: jnp.searchsorted(s, jgrid, side="left").astype(jnp.int32)
    )(slot)
    safe = jnp.minimum(starts[:, :MAXB], BPW - 1)
    blist = jnp.take_along_axis(blocks, safe, axis=1)
    return blist, starts, cols, nblk


def _pack(a, cols128):
    return a.reshape(NW, cols128, 128).astype(jnp.int32)


def kernel(users, items, user_table, item_table):
    users = users.astype(jnp.int32)
    items = items.astype(jnp.int32)

    pos = jnp.arange(B, dtype=jnp.int32)
    vu = jnp.sort(lax.shift_left(lax.shift_right_logical(users, 7), 14) | pos)
    vi = jnp.sort(lax.shift_left(lax.shift_right_logical(items, 7), 14) | pos)
    su = jnp.bitwise_and(vu, B - 1)
    si = jnp.bitwise_and(vi, B - 1)
    users_s = users[su]
    items_s = items[si]

    ubl, ust, ucol, unb = _dedup_prep(users_s)
    ibl, ist, icol, inb = _dedup_prep(items_s)

    uparams = jnp.zeros((NW, 128), jnp.int32).at[:, 0].set(unb)
    iparams = jnp.zeros((NW, 128), jnp.int32).at[:, 0].set(inb)

    ust_p = jnp.concatenate(
        [ust, jnp.full((NW, 5 * 128 - (MAXB + 1)), BPW, jnp.int32)], axis=1)
    ist_p = jnp.concatenate(
        [ist, jnp.full((NW, 5 * 128 - (MAXB + 1)), BPW, jnp.int32)], axis=1)

    uemb = _phase_a(
        _pack(ubl, NCH), _pack(ust_p, NCH + 1), _pack(ucol, NCH),
        uparams.reshape(NW, 1, 128), _pack(su, NCH), user_table.T)
    out = _phase_b(
        _pack(ibl, NCH), _pack(ist_p, NCH + 1), _pack(icol, NCH),
        iparams.reshape(NW, 1, 128), _pack(si, NCH),
        item_table.T, uemb)
    return out


# speedup vs baseline: 1.7464x; 1.7464x over previous
import functools

import jax
import jax.numpy as jnp
from jax import lax
from jax.experimental import pallas as pl
from jax.experimental.pallas import tpu as pltpu
from jax.experimental.pallas import tpu_sc as plsc

B = 16384
D = 64
NC = 2
NS = 16
L = 16
NW = NC * NS
BPW = B // NW
NCH = BPW // 128
RING = 6
MAXB = BPW

_mesh = plsc.VectorSubcoreMesh(
    core_axis_name="c", subcore_axis_name="s", num_cores=NC, num_subcores=NS
)

_cp = pltpu.CompilerParams(needs_layout_passes=False)


def _stage(hbm3, vmem, w):
    pltpu.sync_copy(hbm3.at[w], vmem)


def _vec_at(ref, pos):
    row = lax.shift_right_logical(pos, 7)
    off = jnp.bitwise_and(pos, 127)
    aligned = jnp.bitwise_and(off, ~15)
    return ref[row, pl.ds(pl.multiple_of(aligned, 16), 16)]


def _splat_at(ref, pos):
    grp = _vec_at(ref, pos)
    lane = jnp.bitwise_and(pos, 15)
    lvec = jnp.zeros((L,), jnp.int32) + lane
    return jnp.take_along_axis(grp, lvec, axis=0)


def _scalar_at(ref, pos):
    return _splat_at(ref, pos)[0]


def _dvecs():
    base = lax.iota(jnp.int32, L)
    return [base + k * L for k in range(4)]


@functools.partial(
    pl.kernel,
    mesh=_mesh,
    out_type=jax.ShapeDtypeStruct((B, 128), jnp.float32),
    scratch_types=[
        pltpu.VMEM((NCH, 128), jnp.int32),
        pltpu.VMEM((NCH + 1, 128), jnp.int32),
        pltpu.VMEM((NCH, 128), jnp.int32),
        pltpu.VMEM((1, 128), jnp.int32),
        pltpu.VMEM((NCH, 128), jnp.int32),
        pltpu.VMEM((BPW, 128), jnp.float32),
    ]
    + [pltpu.VMEM((D, 128), jnp.float32) for _ in range(RING)]
    + [pltpu.SemaphoreType.DMA for _ in range(RING + 1)],
    compiler_params=_cp,
)
def _phase_a(blist_h, starts_h, params_h, opos_h, ids_h, tabT_h, emb_out,
             blist_v, starts_v, cols_v, params_v, opos_v, emat_v, *ring):
    bufs, sems, ssem = ring[:RING], ring[RING:-1], ring[-1]
    w = lax.axis_index("s") * NC + lax.axis_index("c")
    _stage(blist_h, blist_v, w)
    _stage(starts_h, starts_v, w)
    _stage(params_h, params_v, w)
    _stage(opos_h, opos_v, w)
    idcopies = [
        pltpu.async_copy(ids_h.at[opos_v.at[j]],
                         cols_v.at[j, pl.ds(0, 128)], ssem)
        for j in range(NCH)]
    for c in idcopies:
        c.wait()
    nblk = params_v[0, pl.ds(0, 16)][0]
    dvecs = _dvecs()

    def fire(j, r):
        ub = _scalar_at(blist_v, j)
        return pltpu.async_copy(
            tabT_h.at[:, pl.ds(pl.multiple_of(ub * 128, 128), 128)],
            bufs[r], sems[r])

    for r in range(RING):
        @pl.when(r < nblk)
        def _(r=r):
            fire(r, r)

    def extract(j, r):
        s = _scalar_at(starts_v, j)
        t = _scalar_at(starts_v, j + 1)

        def elem(e, _):
            csplat = jnp.bitwise_and(_splat_at(cols_v, e), 127)
            for k in range(4):
                v = plsc.load_gather(bufs[r], [dvecs[k], csplat])
                emat_v[e, pl.ds(k * L, L)] = v
            return 0

        lax.fori_loop(s, t, elem, 0)

    def outer(j6, _):
        for r in range(RING):
            j = j6 * RING + r

            @pl.when(j < nblk)
            def _(j=j, r=r):
                pltpu.make_async_copy(
                    tabT_h.at[:, pl.ds(0, 128)], bufs[r], sems[r]).wait()
                extract(j, r)

                @pl.when(j + RING < nblk)
                def _(j=j, r=r):
                    fire(j + RING, r)
        return 0

    nrounds = lax.div(nblk + RING - 1, RING)
    lax.fori_loop(0, nrounds, outer, 0)

    scopies = []
    for j in range(NCH):
        scopies.append(pltpu.async_copy(
            emat_v.at[pl.ds(j * 128, 128)], emb_out.at[opos_v.at[j]], ssem))
    for c in scopies:
        c.wait()


@functools.partial(
    pl.kernel,
    mesh=_mesh,
    out_type=jax.ShapeDtypeStruct((B,), jnp.float32),
    scratch_types=[
        pltpu.VMEM((NCH, 128), jnp.int32),
        pltpu.VMEM((NCH + 1, 128), jnp.int32),
        pltpu.VMEM((NCH, 128), jnp.int32),
        pltpu.VMEM((1, 128), jnp.int32),
        pltpu.VMEM((NCH, 128), jnp.int32),
        pltpu.VMEM((BPW, 128), jnp.float32),
        pltpu.VMEM((BPW,), jnp.float32),
    ]
    + [pltpu.VMEM((D, 128), jnp.float32) for _ in range(RING)]
    + [pltpu.SemaphoreType.DMA for _ in range(RING + 1)],
    compiler_params=_cp,
)
def _phase_b(blist_h, starts_h, params_h, outpos_h, ids_h,
             tabT_h, uemb_h, out_h,
             blist_v, starts_v, cols_v, params_v, outpos_v,
             urows_v, outv, *ring):
    bufs, sems, gsem = ring[:RING], ring[RING:-1], ring[-1]
    w = lax.axis_index("s") * NC + lax.axis_index("c")
    _stage(blist_h, blist_v, w)
    _stage(starts_h, starts_v, w)
    _stage(params_h, params_v, w)
    _stage(outpos_h, outpos_v, w)
    idcopies = [
        pltpu.async_copy(ids_h.at[outpos_v.at[j]],
                         cols_v.at[j, pl.ds(0, 128)], gsem)
        for j in range(NCH)]
    for c in idcopies:
        c.wait()
    nblk = params_v[0, pl.ds(0, 16)][0]
    dvecs = _dvecs()

    def fire(j, r):
        ib = _scalar_at(blist_v, j)
        return pltpu.async_copy(
            tabT_h.at[:, pl.ds(pl.multiple_of(ib * 128, 128), 128)],
            bufs[r], sems[r])

    for r in range(RING):
        @pl.when(r < nblk)
        def _(r=r):
            fire(r, r)

    gcopies = []
    for j in range(NCH):
        gcopies.append(pltpu.async_copy(
            uemb_h.at[outpos_v.at[j]], urows_v.at[pl.ds(j * 128, 128)], gsem))
    for c in gcopies:
        c.wait()

    def extract(j, r):
        s = _scalar_at(starts_v, j)
        t = _scalar_at(starts_v, j + 1)

        def elem(e, _):
            csplat = jnp.bitwise_and(_splat_at(cols_v, e), 127)
            acc = jnp.zeros((L,), jnp.float32)
            for k in range(4):
                iv = plsc.load_gather(bufs[r], [dvecs[k], csplat])
                uv = urows_v[e, pl.ds(k * L, L)]
                acc = acc + iv * uv
            for sh in (8, 4, 2, 1):
                perm = jnp.bitwise_xor(lax.iota(jnp.int32, L), sh)
                acc = acc + jnp.take_along_axis(acc, perm, axis=0)
            aligned = jnp.bitwise_and(e, ~15)
            lane = jnp.bitwise_and(e, 15)
            lmask = lax.iota(jnp.int32, L) == lane
            cur = outv[pl.ds(pl.multiple_of(aligned, 16), L)]
            outv[pl.ds(pl.multiple_of(aligned, 16), L)] = jnp.where(
                lmask, acc, cur)
            return 0

        lax.fori_loop(s, t, elem, 0)

    def outer(j6, _):
        for r in range(RING):
            j = j6 * RING + r

            @pl.when(j < nblk)
            def _(j=j, r=r):
                pltpu.make_async_copy(
                    tabT_h.at[:, pl.ds(0, 128)], bufs[r], sems[r]).wait()
                extract(j, r)

                @pl.when(j + RING < nblk)
                def _(j=j, r=r):
                    fire(j + RING, r)
        return 0

    nrounds = lax.div(nblk + RING - 1, RING)
    lax.fori_loop(0, nrounds, outer, 0)

    scopies = []
    for j in range(NCH):
        scopies.append(pltpu.async_copy(
            outv.at[pl.ds(j * 128, 128)], out_h.at[outpos_v.at[j]], gsem))
    for c in scopies:
        c.wait()


def _dedup_prep(blocks_sorted):
    blocks = blocks_sorted.reshape(NW, BPW)
    first = jnp.concatenate(
        [jnp.ones((NW, 1), bool), blocks[:, 1:] != blocks[:, :-1]], axis=1)
    slot = jnp.cumsum(first.astype(jnp.int32), axis=1) - 1
    nblk = slot[:, -1] + 1
    jgrid = jnp.arange(MAXB + 1, dtype=jnp.int32)
    lt = slot[:, None, :] < jgrid[None, :, None]
    starts = jnp.sum(lt, axis=2, dtype=jnp.int32)
    safe = jnp.minimum(starts[:, :MAXB], BPW - 1)
    blist = jnp.take_along_axis(blocks, safe, axis=1)
    return blist, starts, nblk


def _pack(a, cols128):
    return a.reshape(NW, cols128, 128).astype(jnp.int32)


def kernel(users, items, user_table, item_table):
    users = users.astype(jnp.int32)
    items = items.astype(jnp.int32)

    pos = jnp.arange(B, dtype=jnp.int32)
    vu = jnp.sort(lax.shift_left(lax.shift_right_logical(users, 7), 14) | pos)
    vi = jnp.sort(lax.shift_left(lax.shift_right_logical(items, 7), 14) | pos)
    su = jnp.bitwise_and(vu, B - 1)
    si = jnp.bitwise_and(vi, B - 1)

    ubl, ust, unb = _dedup_prep(lax.shift_right_logical(vu, 14))
    ibl, ist, inb = _dedup_prep(lax.shift_right_logical(vi, 14))

    uparams = jnp.zeros((NW, 128), jnp.int32).at[:, 0].set(unb)
    iparams = jnp.zeros((NW, 128), jnp.int32).at[:, 0].set(inb)

    ust_p = jnp.concatenate(
        [ust, jnp.full((NW, 5 * 128 - (MAXB + 1)), BPW, jnp.int32)], axis=1)
    ist_p = jnp.concatenate(
        [ist, jnp.full((NW, 5 * 128 - (MAXB + 1)), BPW, jnp.int32)], axis=1)

    uemb = _phase_a(
        _pack(ubl, NCH), _pack(ust_p, NCH + 1),
        uparams.reshape(NW, 1, 128), _pack(su, NCH), users, user_table.T)
    out = _phase_b(
        _pack(ibl, NCH), _pack(ist_p, NCH + 1),
        iparams.reshape(NW, 1, 128), _pack(si, NCH), items,
        item_table.T, uemb)
    return out
